# Initial kernel scaffold; baseline (speedup 1.0000x reference)
import functools

import jax
import jax.numpy as jnp
from jax import lax
from jax.experimental import pallas as pl
from jax.experimental.pallas import tpu as pltpu
from jax.experimental.pallas import tpu_sc as plsc

NC = 2
NS = 16
NW = NC * NS
CHUNK = 128
JPC = 14
EDGE_QUANTUM = NW * CHUNK * JPC


def _gather_body(rw, x_hbm, idx_hbm, out_hbm, idx_v, rows_v, sem):
    c = lax.axis_index("c")
    s = lax.axis_index("s")
    wid = s * NC + c
    row0 = wid * rw

    def outer(i, carry):
        r = row0 + i * JPC
        pltpu.sync_copy(idx_hbm.at[pl.ds(r, JPC)], idx_v)
        copies = []
        for j in range(JPC):
            copies.append(
                pltpu.async_copy(
                    x_hbm.at[idx_v.at[j]],
                    rows_v.at[pl.ds(j * CHUNK, CHUNK)],
                    sem,
                )
            )
        for cp in copies:
            cp.wait()
        pltpu.sync_copy(rows_v, out_hbm.at[pl.ds(r * CHUNK, JPC * CHUNK)])
        return carry

    lax.fori_loop(0, rw // JPC, outer, 0)


def _scatter_body(n, rw, msg_hbm, dst_hbm, zeros_hbm, out_hbm, idx_v, msg_v, acc):
    c = lax.axis_index("c")
    s = lax.axis_index("s")
    wid = s * NC + c
    zr = n // NS

    pltpu.sync_copy(zeros_hbm.at[pl.ds(s * zr, zr)], acc.at[pl.ds(s * zr, zr)])
    plsc.subcore_barrier()

    def outer(i, carry):
        r = wid * rw + i * JPC
        pltpu.sync_copy(dst_hbm.at[pl.ds(r, JPC)], idx_v)
        pltpu.sync_copy(msg_hbm.at[pl.ds(r * CHUNK, JPC * CHUNK)], msg_v)
        for j in range(JPC):
            pltpu.sync_copy(
                msg_v.at[pl.ds(j * CHUNK, CHUNK)],
                acc.at[idx_v.at[j]],
                add=True,
            )
        return carry

    lax.fori_loop(0, rw // JPC, outer, 0)
    plsc.subcore_barrier()
    pltpu.sync_copy(acc.at[pl.ds(s * zr, zr)], out_hbm.at[c, pl.ds(s * zr, zr)])


def _msg_body(e, bt, ea_ref, xj_ref, w1_ref, b1_ref, w2_ref, b2_ref,
              r_ref, s_ref, a1_ref, out_ref):
    a1 = a1_ref[0, 0]
    h = jnp.dot(ea_ref[...], w1_ref[...], preferred_element_type=jnp.float32)
    h = h + b1_ref[0]
    h = jnp.where(h >= 0, h, a1 * h)
    theta = jnp.dot(h, w2_ref[...], preferred_element_type=jnp.float32)
    theta = theta + b2_ref[0]
    xrep = jnp.dot(xj_ref[...], r_ref[...], preferred_element_type=jnp.float32)
    msg = jnp.dot(xrep * theta, s_ref[...], preferred_element_type=jnp.float32)
    gid = pl.program_id(0) * bt + lax.broadcasted_iota(jnp.int32, msg.shape, 0)
    out_ref[...] = jnp.where(gid < e, msg, 0.0)


def _final_body(p0_ref, p1_ref, x_ref, root_ref, bias_ref, a2_ref, out_ref):
    a2 = a2_ref[0, 0]
    v = (p0_ref[...] + p1_ref[...]
         + jnp.dot(x_ref[...], root_ref[...], preferred_element_type=jnp.float32)
         + bias_ref[0])
    out_ref[...] = jnp.where(v >= 0, v, a2 * v)


def kernel(x, edge_index, edge_attr, W1, b1, a1, W2, b2, root, bias, a2):
    n, d = x.shape
    e = edge_index.shape[1]
    de = edge_attr.shape[1]
    f32 = jnp.float32

    e_pad = ((e + EDGE_QUANTUM - 1) // EDGE_QUANTUM) * EDGE_QUANTUM
    pad = e_pad - e
    src = edge_index[0].astype(jnp.int32)
    dst = edge_index[1].astype(jnp.int32)
    src_p = jnp.concatenate([src, jnp.zeros((pad,), jnp.int32)])
    dst_p = jnp.concatenate([dst, jnp.zeros((pad,), jnp.int32)])
    ea_p = jnp.concatenate([edge_attr.astype(f32), jnp.zeros((pad, de), f32)])
    src2d = src_p.reshape(-1, CHUNK)
    dst2d = dst_p.reshape(-1, CHUNK)
    rw = e_pad // (NW * CHUNK)

    ii = jnp.arange(d, dtype=jnp.int32)
    oo = jnp.arange(d * d, dtype=jnp.int32)
    rmat = (ii[:, None] == (oo[None, :] // d)).astype(f32)
    smat = ((oo[:, None] % d) == ii[None, :]).astype(f32)

    mesh = plsc.VectorSubcoreMesh(core_axis_name="c", subcore_axis_name="s")
    gather = pl.kernel(
        functools.partial(_gather_body, rw),
        out_type=jax.ShapeDtypeStruct((e_pad, d), f32),
        mesh=mesh,
        scratch_types=[
            pltpu.VMEM((JPC, CHUNK), jnp.int32),
            pltpu.VMEM((JPC * CHUNK, d), f32),
            pltpu.SemaphoreType.DMA,
        ],
    )
    x_j = gather(x.astype(f32), src2d)

    bt = 4096
    grid = e_pad // bt
    msg = pl.pallas_call(
        functools.partial(_msg_body, e, bt),
        grid=(grid,),
        in_specs=[
            pl.BlockSpec((bt, de), lambda b: (b, 0)),
            pl.BlockSpec((bt, d), lambda b: (b, 0)),
            pl.BlockSpec((de, d), lambda b: (0, 0)),
            pl.BlockSpec((1, d), lambda b: (0, 0)),
            pl.BlockSpec((d, d * d), lambda b: (0, 0)),
            pl.BlockSpec((1, d * d), lambda b: (0, 0)),
            pl.BlockSpec((d, d * d), lambda b: (0, 0)),
            pl.BlockSpec((d * d, d), lambda b: (0, 0)),
            pl.BlockSpec(memory_space=pltpu.MemorySpace.SMEM),
        ],
        out_specs=pl.BlockSpec((bt, d), lambda b: (b, 0)),
        out_shape=jax.ShapeDtypeStruct((e_pad, d), f32),
    )(ea_p, x_j,
      W1.astype(f32), b1.astype(f32).reshape(1, d),
      W2.astype(f32), b2.astype(f32).reshape(1, d * d),
      rmat, smat,
      jnp.asarray(a1, f32).reshape(1, 1))

    scatter = pl.kernel(
        functools.partial(_scatter_body, n, rw),
        out_type=jax.ShapeDtypeStruct((NC, n, d), f32),
        mesh=mesh,
        scratch_types=[
            pltpu.VMEM((JPC, CHUNK), jnp.int32),
            pltpu.VMEM((JPC * CHUNK, d), f32),
            pltpu.VMEM_SHARED((n, d), f32),
        ],
    )
    partials = scatter(msg, dst2d, jnp.zeros((n, d), f32))

    bn = 2000
    out = pl.pallas_call(
        _final_body,
        grid=(n // bn,),
        in_specs=[
            pl.BlockSpec((bn, d), lambda b: (b, 0)),
            pl.BlockSpec((bn, d), lambda b: (b, 0)),
            pl.BlockSpec((bn, d), lambda b: (b, 0)),
            pl.BlockSpec((d, d), lambda b: (0, 0)),
            pl.BlockSpec((1, d), lambda b: (0, 0)),
            pl.BlockSpec(memory_space=pltpu.MemorySpace.SMEM),
        ],
        out_specs=pl.BlockSpec((bn, d), lambda b: (b, 0)),
        out_shape=jax.ShapeDtypeStruct((n, d), f32),
    )(partials[0], partials[1], x.astype(f32),
      root.astype(f32), bias.astype(f32).reshape(1, d),
      jnp.asarray(a2, f32).reshape(1, 1))
    return out


# speedup vs baseline: 9.4566x; 9.4566x over previous
import functools

import jax
import jax.numpy as jnp
from jax import lax
from jax.experimental import pallas as pl
from jax.experimental.pallas import tpu as pltpu
from jax.experimental.pallas import tpu_sc as plsc

NC = 2
NS = 16
NW = NC * NS
CHUNK = 128
BT = 4096
NB = 4096


def _pick_jpc(rw):
    for j in (14, 8, 7, 4, 2, 1):
        if rw % j == 0:
            return j
    return 1


def _gather_body(rw, jpc, x_hbm, idx_hbm, out_hbm, idx_v, rows_v, sem):
    c = lax.axis_index("c")
    s = lax.axis_index("s")
    wid = s * NC + c
    row0 = wid * rw

    def outer(i, carry):
        r = row0 + i * jpc
        pltpu.sync_copy(idx_hbm.at[pl.ds(r, jpc)], idx_v)
        copies = []
        for j in range(jpc):
            copies.append(
                pltpu.async_copy(
                    x_hbm.at[idx_v.at[j]],
                    rows_v.at[pl.ds(j * CHUNK, CHUNK)],
                    sem,
                )
            )
        for cp in copies:
            cp.wait()
        pltpu.sync_copy(rows_v, out_hbm.at[pl.ds(r * CHUNK, jpc * CHUNK)])
        return carry

    lax.fori_loop(0, rw // jpc, outer, 0)


def _scatter_body(n_pad, rw, jpc, msg_hbm, dst_hbm, zeros_hbm, out0_hbm,
                  out1_hbm, idx_v, msg_v, acc):
    c = lax.axis_index("c")
    s = lax.axis_index("s")
    wid = s * NC + c
    zr = n_pad // NS

    pltpu.sync_copy(zeros_hbm.at[pl.ds(s * zr, zr)], acc.at[pl.ds(s * zr, zr)])
    plsc.subcore_barrier()

    def outer(i, carry):
        r = wid * rw + i * jpc
        pltpu.sync_copy(dst_hbm.at[pl.ds(r, jpc)], idx_v)
        pltpu.sync_copy(msg_hbm.at[pl.ds(r * CHUNK, jpc * CHUNK)], msg_v)
        for j in range(jpc):
            pltpu.sync_copy(
                msg_v.at[pl.ds(j * CHUNK, CHUNK)],
                acc.at[idx_v.at[j]],
                add=True,
            )
        return carry

    lax.fori_loop(0, rw // jpc, outer, 0)
    plsc.subcore_barrier()

    @pl.when(c == 0)
    def _():
        pltpu.sync_copy(acc.at[pl.ds(s * zr, zr)], out0_hbm.at[pl.ds(s * zr, zr)])

    @pl.when(c == 1)
    def _():
        pltpu.sync_copy(acc.at[pl.ds(s * zr, zr)], out1_hbm.at[pl.ds(s * zr, zr)])


def _unpack_to_featmajor(packed):
    a = packed.T
    return jnp.concatenate([a[16 * k:16 * (k + 1), :] for k in range(8)], axis=1)


def _pack_from_featmajor(fm):
    w = fm.shape[1] // 8
    return jnp.concatenate(
        [fm[:, w * k:w * (k + 1)].T for k in range(8)], axis=1)


def _msg_body(ea_t_ref, xjp_ref, w1t_ref, b1_ref, w2f_ref, b2m_ref,
              a1_ref, out_ref):
    a1 = a1_ref[0, 0]
    f32 = jnp.float32
    bf16 = jnp.bfloat16
    d = ea_t_ref.shape[0]
    ht = jnp.dot(w1t_ref[...], ea_t_ref[...], preferred_element_type=f32)
    ht = ht + b1_ref[...]
    ht = jnp.where(ht >= 0, ht, a1 * ht)
    xj_t = _unpack_to_featmajor(xjp_ref[...])
    ht16 = ht.astype(bf16)
    x16 = xj_t.astype(bf16)
    zh = jnp.concatenate([ht16] * d, axis=0)
    zx = jnp.concatenate(
        [jnp.broadcast_to(x16[i:i + 1, :], ht16.shape) for i in range(d)],
        axis=0)
    msg_t = (jnp.dot(w2f_ref[...], zh * zx, preferred_element_type=f32)
             + jnp.dot(b2m_ref[...], xj_t, preferred_element_type=f32))
    out_ref[...] = _pack_from_featmajor(msg_t)


def _final_body(p0_ref, p1_ref, xt_ref, roott_ref, bias_ref, a2_ref, out_ref):
    a2 = a2_ref[0, 0]
    xr_t = jnp.dot(roott_ref[...], xt_ref[...],
                   preferred_element_type=jnp.float32)
    v = p0_ref[...] + p1_ref[...] + _pack_from_featmajor(xr_t) + bias_ref[...]
    out_ref[...] = jnp.where(v >= 0, v, a2 * v)


def kernel(x, edge_index, edge_attr, W1, b1, a1, W2, b2, root, bias, a2):
    n, d = x.shape
    e = edge_index.shape[1]
    de = edge_attr.shape[1]
    f32 = jnp.float32

    e_pad = ((e + BT - 1) // BT) * BT
    n_pad = ((n + NB - 1) // NB) * NB
    pad = e_pad - e
    src = edge_index[0].astype(jnp.int32)
    dst = edge_index[1].astype(jnp.int32)
    dblk, dloc = dst // NB, dst % NB
    dst_row = dblk * NB + (dloc % 512) * 8 + dloc // 512
    src_p = jnp.concatenate([src, jnp.zeros((pad,), jnp.int32)])
    mloc = (n_pad - 1) % NB
    dump_row = jnp.int32((n_pad - 1) // NB * NB + (mloc % 512) * 8 + mloc // 512)
    dst_p = jnp.concatenate([dst_row, jnp.full((pad,), dump_row, jnp.int32)])
    idx_g = src_p.reshape(-1, 8, 512).swapaxes(1, 2).reshape(-1, CHUNK)
    dst_g = dst_p.reshape(-1, 8, 512).swapaxes(1, 2).reshape(-1, CHUNK)
    rw = e_pad // (NW * CHUNK)
    jpc = _pick_jpc(rw)

    ea_t = edge_attr.astype(f32).T
    x_t = x.astype(f32).T
    w1_t = W1.astype(f32).T
    bf16 = jnp.bfloat16
    w2f = W2.astype(f32).reshape(d, d, d).transpose(2, 1, 0).reshape(
        d, d * d).astype(bf16)
    b2m = b2.astype(f32).reshape(d, d).T
    root_t = root.astype(f32).T
    bias128 = jnp.tile(bias.astype(f32), 8).reshape(1, 8 * d)

    mesh = plsc.VectorSubcoreMesh(core_axis_name="c", subcore_axis_name="s")
    gather = pl.kernel(
        functools.partial(_gather_body, rw, jpc),
        out_type=jax.ShapeDtypeStruct((e_pad, d), f32),
        mesh=mesh,
        scratch_types=[
            pltpu.VMEM((jpc, CHUNK), jnp.int32),
            pltpu.VMEM((jpc * CHUNK, d), f32),
            pltpu.SemaphoreType.DMA,
        ],
        compiler_params=pltpu.CompilerParams(use_tc_tiling_on_sc=False),
    )
    x_j = gather(x.astype(f32), idx_g)
    xjp = x_j.reshape(e_pad // 8, 8 * d)

    msgp = pl.pallas_call(
        _msg_body,
        grid=(e_pad // BT,),
        in_specs=[
            pl.BlockSpec((de, BT), lambda b: (0, b)),
            pl.BlockSpec((BT // 8, 8 * d), lambda b: (b, 0)),
            pl.BlockSpec((d, de), lambda b: (0, 0)),
            pl.BlockSpec((d, 1), lambda b: (0, 0)),
            pl.BlockSpec((d, d * d), lambda b: (0, 0)),
            pl.BlockSpec((d, d), lambda b: (0, 0)),
            pl.BlockSpec(memory_space=pltpu.MemorySpace.SMEM),
        ],
        out_specs=pl.BlockSpec((BT // 8, 8 * d), lambda b: (b, 0)),
        out_shape=jax.ShapeDtypeStruct((e_pad // 8, 8 * d), f32),
    )(ea_t, xjp, w1_t, b1.astype(f32).reshape(d, 1), w2f, b2m,
      jnp.asarray(a1, f32).reshape(1, 1))
    msg_lin = msgp.reshape(e_pad, d)

    scatter = pl.kernel(
        functools.partial(_scatter_body, n_pad, rw, jpc),
        out_type=(jax.ShapeDtypeStruct((n_pad, d), f32),
                  jax.ShapeDtypeStruct((n_pad, d), f32)),
        mesh=mesh,
        scratch_types=[
            pltpu.VMEM((jpc, CHUNK), jnp.int32),
            pltpu.VMEM((jpc * CHUNK, d), f32),
            pltpu.VMEM_SHARED((n_pad, d), f32),
        ],
        compiler_params=pltpu.CompilerParams(use_tc_tiling_on_sc=False),
    )
    p0, p1 = scatter(msg_lin, dst_g, jnp.zeros((n_pad, d), f32))
    p0p = p0.reshape(n_pad // 8, 8 * d)
    p1p = p1.reshape(n_pad // 8, 8 * d)

    outp = pl.pallas_call(
        _final_body,
        grid=(n_pad // NB,),
        in_specs=[
            pl.BlockSpec((NB // 8, 8 * d), lambda b: (b, 0)),
            pl.BlockSpec((NB // 8, 8 * d), lambda b: (b, 0)),
            pl.BlockSpec((d, NB), lambda b: (0, b)),
            pl.BlockSpec((d, d), lambda b: (0, 0)),
            pl.BlockSpec((1, 8 * d), lambda b: (0, 0)),
            pl.BlockSpec(memory_space=pltpu.MemorySpace.SMEM),
        ],
        out_specs=pl.BlockSpec((NB // 8, 8 * d), lambda b: (b, 0)),
        out_shape=jax.ShapeDtypeStruct((n_pad // 8, 8 * d), f32),
    )(p0p, p1p, x_t, root_t, bias128, jnp.asarray(a2, f32).reshape(1, 1))

    out = outp.reshape(n_pad // NB, 512, 8, d).swapaxes(1, 2).reshape(-1, d)
    return out[:n]


# speedup vs baseline: 12.8469x; 1.3585x over previous
import functools

import jax
import jax.numpy as jnp
from jax import lax
from jax.experimental import pallas as pl
from jax.experimental.pallas import tpu as pltpu
from jax.experimental.pallas import tpu_sc as plsc

NC = 2
NS = 16
NW = NC * NS
CHUNK = 128
BT = 8192
NB = 4096


def _pick_jpc(rw):
    for j in (14, 8, 7, 4, 2, 1):
        if rw % j == 0:
            return j
    return 1


def _gather_body(rw, jpc, x_hbm, idx_hbm, out_hbm, idx_v, rows_v, sem):
    c = lax.axis_index("c")
    s = lax.axis_index("s")
    wid = s * NC + c
    row0 = wid * rw

    def outer(i, carry):
        r = row0 + i * jpc
        pltpu.sync_copy(idx_hbm.at[pl.ds(r, jpc)], idx_v)
        copies = []
        for j in range(jpc):
            copies.append(
                pltpu.async_copy(
                    x_hbm.at[idx_v.at[j]],
                    rows_v.at[pl.ds(j * CHUNK, CHUNK)],
                    sem,
                )
            )
        for cp in copies:
            cp.wait()
        pltpu.sync_copy(rows_v, out_hbm.at[pl.ds(r * CHUNK, jpc * CHUNK)])
        return carry

    lax.fori_loop(0, rw // jpc, outer, 0)


def _scatter_body(n_pad, rw, jpc, msg_hbm, dst_hbm, zeros_hbm, out0_hbm,
                  out1_hbm, idx_v, msg_v, acc):
    c = lax.axis_index("c")
    s = lax.axis_index("s")
    wid = s * NC + c
    zr = n_pad // NS

    pltpu.sync_copy(zeros_hbm.at[pl.ds(s * zr, zr)], acc.at[pl.ds(s * zr, zr)])
    plsc.subcore_barrier()

    def outer(i, carry):
        r = wid * rw + i * jpc
        pltpu.sync_copy(dst_hbm.at[pl.ds(r, jpc)], idx_v)
        pltpu.sync_copy(msg_hbm.at[pl.ds(r * CHUNK, jpc * CHUNK)], msg_v)
        for j in range(jpc):
            pltpu.sync_copy(
                msg_v.at[pl.ds(j * CHUNK, CHUNK)],
                acc.at[idx_v.at[j]],
                add=True,
            )
        return carry

    lax.fori_loop(0, rw // jpc, outer, 0)
    plsc.subcore_barrier()

    @pl.when(c == 0)
    def _():
        pltpu.sync_copy(acc.at[pl.ds(s * zr, zr)], out0_hbm.at[pl.ds(s * zr, zr)])

    @pl.when(c == 1)
    def _():
        pltpu.sync_copy(acc.at[pl.ds(s * zr, zr)], out1_hbm.at[pl.ds(s * zr, zr)])


def _unpack_to_featmajor(packed):
    a = packed.T
    return jnp.concatenate([a[16 * k:16 * (k + 1), :] for k in range(8)], axis=1)


def _pack_from_featmajor(fm):
    w = fm.shape[1] // 8
    stacked = jnp.concatenate(
        [fm[:, w * k:w * (k + 1)] for k in range(8)], axis=0)
    return stacked.T


def _msg_body(ea_t_ref, xjp_ref, w1t_ref, b1_ref, w2f_ref, b2m_ref,
              a1_ref, out_ref):
    a1 = a1_ref[0, 0]
    f32 = jnp.float32
    bf16 = jnp.bfloat16
    d = ea_t_ref.shape[0]
    ht = jnp.dot(w1t_ref[...], ea_t_ref[...], preferred_element_type=f32)
    ht = ht + b1_ref[...]
    ht = jnp.where(ht >= 0, ht, a1 * ht)
    xj_t = _unpack_to_featmajor(xjp_ref[...])
    ht16 = ht.astype(bf16)
    x16 = xj_t.astype(bf16)
    zh = jnp.concatenate([ht16] * d, axis=0)
    zx = jnp.concatenate(
        [jnp.broadcast_to(x16[i:i + 1, :], ht16.shape) for i in range(d)],
        axis=0)
    msg_t = (jnp.dot(w2f_ref[...], zh * zx, preferred_element_type=f32)
             + jnp.dot(b2m_ref[...], xj_t, preferred_element_type=f32))
    out_ref[...] = _pack_from_featmajor(msg_t)


def _final_body(p0_ref, p1_ref, xt_ref, roott_ref, bias_ref, a2_ref, out_ref):
    a2 = a2_ref[0, 0]
    xr_t = jnp.dot(roott_ref[...], xt_ref[...],
                   preferred_element_type=jnp.float32)
    v = p0_ref[...] + p1_ref[...] + _pack_from_featmajor(xr_t) + bias_ref[...]
    out_ref[...] = jnp.where(v >= 0, v, a2 * v)


def kernel(x, edge_index, edge_attr, W1, b1, a1, W2, b2, root, bias, a2):
    n, d = x.shape
    e = edge_index.shape[1]
    de = edge_attr.shape[1]
    f32 = jnp.float32

    e_pad = ((e + BT - 1) // BT) * BT
    n_pad = ((n + NB - 1) // NB) * NB
    pad = e_pad - e
    src = edge_index[0].astype(jnp.int32)
    dst = edge_index[1].astype(jnp.int32)
    wn = NB // 8
    dblk, dloc = dst // NB, dst % NB
    dst_row = dblk * NB + (dloc % wn) * 8 + dloc // wn
    src_p = jnp.concatenate([src, jnp.zeros((pad,), jnp.int32)])
    mloc = (n_pad - 1) % NB
    dump_row = jnp.int32((n_pad - 1) // NB * NB + (mloc % wn) * 8 + mloc // wn)
    dst_p = jnp.concatenate([dst_row, jnp.full((pad,), dump_row, jnp.int32)])
    we = BT // 8
    idx_g = src_p.reshape(-1, 8, we).swapaxes(1, 2).reshape(-1, CHUNK)
    dst_g = dst_p.reshape(-1, 8, we).swapaxes(1, 2).reshape(-1, CHUNK)
    rw = e_pad // (NW * CHUNK)
    jpc = _pick_jpc(rw)

    ea_t = edge_attr.astype(f32).T
    x_t = x.astype(f32).T
    w1_t = W1.astype(f32).T
    bf16 = jnp.bfloat16
    w2f = W2.astype(f32).reshape(d, d, d).transpose(2, 1, 0).reshape(
        d, d * d).astype(bf16)
    b2m = b2.astype(f32).reshape(d, d).T
    root_t = root.astype(f32).T
    bias128 = jnp.tile(bias.astype(f32), 8).reshape(1, 8 * d)

    mesh = plsc.VectorSubcoreMesh(core_axis_name="c", subcore_axis_name="s")
    gather = pl.kernel(
        functools.partial(_gather_body, rw, jpc),
        out_type=jax.ShapeDtypeStruct((e_pad, d), f32),
        mesh=mesh,
        scratch_types=[
            pltpu.VMEM((jpc, CHUNK), jnp.int32),
            pltpu.VMEM((jpc * CHUNK, d), f32),
            pltpu.SemaphoreType.DMA,
        ],
        compiler_params=pltpu.CompilerParams(use_tc_tiling_on_sc=False),
    )
    x_j = gather(x.astype(f32), idx_g)
    xjp = x_j.reshape(e_pad // 8, 8 * d)

    msgp = pl.pallas_call(
        _msg_body,
        grid=(e_pad // BT,),
        in_specs=[
            pl.BlockSpec((de, BT), lambda b: (0, b)),
            pl.BlockSpec((BT // 8, 8 * d), lambda b: (b, 0)),
            pl.BlockSpec((d, de), lambda b: (0, 0)),
            pl.BlockSpec((d, 1), lambda b: (0, 0)),
            pl.BlockSpec((d, d * d), lambda b: (0, 0)),
            pl.BlockSpec((d, d), lambda b: (0, 0)),
            pl.BlockSpec(memory_space=pltpu.MemorySpace.SMEM),
        ],
        out_specs=pl.BlockSpec((BT // 8, 8 * d), lambda b: (b, 0)),
        out_shape=jax.ShapeDtypeStruct((e_pad // 8, 8 * d), f32),
    )(ea_t, xjp, w1_t, b1.astype(f32).reshape(d, 1), w2f, b2m,
      jnp.asarray(a1, f32).reshape(1, 1))
    msg_lin = msgp.reshape(e_pad, d)

    scatter = pl.kernel(
        functools.partial(_scatter_body, n_pad, rw, jpc),
        out_type=(jax.ShapeDtypeStruct((n_pad, d), f32),
                  jax.ShapeDtypeStruct((n_pad, d), f32)),
        mesh=mesh,
        scratch_types=[
            pltpu.VMEM((jpc, CHUNK), jnp.int32),
            pltpu.VMEM((jpc * CHUNK, d), f32),
            pltpu.VMEM_SHARED((n_pad, d), f32),
        ],
        compiler_params=pltpu.CompilerParams(use_tc_tiling_on_sc=False),
    )
    p0, p1 = scatter(msg_lin, dst_g, jnp.zeros((n_pad, d), f32))
    p0p = p0.reshape(n_pad // 8, 8 * d)
    p1p = p1.reshape(n_pad // 8, 8 * d)

    outp = pl.pallas_call(
        _final_body,
        grid=(n_pad // NB,),
        in_specs=[
            pl.BlockSpec((NB // 8, 8 * d), lambda b: (b, 0)),
            pl.BlockSpec((NB // 8, 8 * d), lambda b: (b, 0)),
            pl.BlockSpec((d, NB), lambda b: (0, b)),
            pl.BlockSpec((d, d), lambda b: (0, 0)),
            pl.BlockSpec((1, 8 * d), lambda b: (0, 0)),
            pl.BlockSpec(memory_space=pltpu.MemorySpace.SMEM),
        ],
        out_specs=pl.BlockSpec((NB // 8, 8 * d), lambda b: (b, 0)),
        out_shape=jax.ShapeDtypeStruct((n_pad // 8, 8 * d), f32),
    )(p0p, p1p, x_t, root_t, bias128, jnp.asarray(a2, f32).reshape(1, 1))

    out = outp.reshape(n_pad // NB, NB // 8, 8, d).swapaxes(1, 2).reshape(-1, d)
    return out[:n]


# speedup vs baseline: 14.3945x; 1.1205x over previous
import functools

import jax
import jax.numpy as jnp
from jax import lax
from jax.experimental import pallas as pl
from jax.experimental.pallas import tpu as pltpu
from jax.experimental.pallas import tpu_sc as plsc

NC = 2
NS = 16
NW = NC * NS
QS = 4
CHUNK = 128
BT = 8192
NB = 4096


def _pick_jpc(rw):
    for j in (14, 8, 7, 4, 2, 1):
        if rw % j == 0:
            return j
    return 1


def _gather_body(rw, jpc, d, x_hbm, idx_hbm, out_hbm, idx_v, rows_v, sem):
    c = lax.axis_index("c")
    s = lax.axis_index("s")
    wid = s * NC + c
    k = wid // QS
    q = wid % QS
    rps_rows = idx_hbm.shape[0] // 8
    row0 = rps_rows * k + rw * q
    rout0 = rw * CHUNK * q

    def outer(i, carry):
        r = row0 + i * jpc
        pltpu.sync_copy(idx_hbm.at[pl.ds(r, jpc)], idx_v)
        copies = []
        for j in range(jpc):
            copies.append(
                pltpu.async_copy(
                    x_hbm.at[idx_v.at[j]],
                    rows_v.at[pl.ds(j * CHUNK, CHUNK)],
                    sem,
                )
            )
        for cp in copies:
            cp.wait()
        pltpu.sync_copy(
            rows_v,
            out_hbm.at[pl.ds(rout0 + i * jpc * CHUNK, jpc * CHUNK),
                       pl.ds(d * k, d)],
        )
        return carry

    lax.fori_loop(0, rw // jpc, outer, 0)


def _scatter_body(n_pad, rw, jpc, d, msg_hbm, dst_hbm, zeros_hbm, out0_hbm,
                  out1_hbm, idx_v, msg_v, acc):
    c = lax.axis_index("c")
    s = lax.axis_index("s")
    wid = s * NC + c
    k = wid // QS
    q = wid % QS
    rps_rows = dst_hbm.shape[0] // 8
    row0 = rps_rows * k + rw * q
    rin0 = rw * CHUNK * q
    zr = n_pad // NS

    pltpu.sync_copy(zeros_hbm.at[pl.ds(s * zr, zr)], acc.at[pl.ds(s * zr, zr)])
    plsc.subcore_barrier()

    def outer(i, carry):
        pltpu.sync_copy(dst_hbm.at[pl.ds(row0 + i * jpc, jpc)], idx_v)
        pltpu.sync_copy(
            msg_hbm.at[pl.ds(rin0 + i * jpc * CHUNK, jpc * CHUNK),
                       pl.ds(d * k, d)],
            msg_v,
        )
        for j in range(jpc):
            pltpu.sync_copy(
                msg_v.at[pl.ds(j * CHUNK, CHUNK)],
                acc.at[idx_v.at[j]],
                add=True,
            )
        return carry

    lax.fori_loop(0, rw // jpc, outer, 0)
    plsc.subcore_barrier()

    @pl.when(c == 0)
    def _():
        pltpu.sync_copy(acc.at[pl.ds(s * zr, zr)], out0_hbm.at[pl.ds(s * zr, zr)])

    @pl.when(c == 1)
    def _():
        pltpu.sync_copy(acc.at[pl.ds(s * zr, zr)], out1_hbm.at[pl.ds(s * zr, zr)])


def _unpack_to_featmajor(packed):
    a = packed.T
    return jnp.concatenate([a[16 * k:16 * (k + 1), :] for k in range(8)], axis=1)


def _pack_from_featmajor(fm):
    w = fm.shape[1] // 8
    stacked = jnp.concatenate(
        [fm[:, w * k:w * (k + 1)] for k in range(8)], axis=0)
    return stacked.T


def _msg_body(ea_refs, xjp_ref, w1t_ref, b1_ref, w2f_ref, b2m_ref,
              a1_ref, out_ref):
    a1 = a1_ref[0, 0]
    f32 = jnp.float32
    bf16 = jnp.bfloat16
    ea_t = jnp.concatenate([r[...] for r in ea_refs], axis=1)
    d = ea_t.shape[0]
    ht = jnp.dot(w1t_ref[...], ea_t, preferred_element_type=f32)
    ht = ht + b1_ref[...]
    ht = jnp.where(ht >= 0, ht, a1 * ht)
    xj_t = _unpack_to_featmajor(xjp_ref[...])
    ht16 = ht.astype(bf16)
    x16 = xj_t.astype(bf16)
    zh = jnp.concatenate([ht16] * d, axis=0)
    zx = jnp.concatenate(
        [jnp.broadcast_to(x16[i:i + 1, :], ht16.shape) for i in range(d)],
        axis=0)
    msg_t = (jnp.dot(w2f_ref[...], zh * zx, preferred_element_type=f32)
             + jnp.dot(b2m_ref[...], xj_t, preferred_element_type=f32))
    out_ref[...] = _pack_from_featmajor(msg_t)


def _msg_body_flat(*refs):
    return _msg_body(refs[:8], *refs[8:])


def _final_body(p0_ref, p1_ref, xt_ref, roott_ref, bias_ref, a2_ref, out_ref):
    a2 = a2_ref[0, 0]
    xr_t = jnp.dot(roott_ref[...], xt_ref[...],
                   preferred_element_type=jnp.float32)
    v = p0_ref[...] + p1_ref[...] + _pack_from_featmajor(xr_t) + bias_ref[...]
    out_ref[...] = jnp.where(v >= 0, v, a2 * v)


def kernel(x, edge_index, edge_attr, W1, b1, a1, W2, b2, root, bias, a2):
    n, d = x.shape
    e = edge_index.shape[1]
    de = edge_attr.shape[1]
    f32 = jnp.float32

    e_pad = ((e + BT - 1) // BT) * BT
    n_pad = ((n + NB - 1) // NB) * NB
    pad = e_pad - e
    src = edge_index[0].astype(jnp.int32)
    dst = edge_index[1].astype(jnp.int32)
    wn = NB // 8
    dblk, dloc = dst // NB, dst % NB
    dst_row = dblk * NB + (dloc % wn) * 8 + dloc // wn
    src_p = jnp.concatenate([src, jnp.zeros((pad,), jnp.int32)])
    mloc = (n_pad - 1) % NB
    dump_row = jnp.int32((n_pad - 1) // NB * NB + (mloc % wn) * 8 + mloc // wn)
    dst_p = jnp.concatenate([dst_row, jnp.full((pad,), dump_row, jnp.int32)])
    src2d = src_p.reshape(-1, CHUNK)
    dst2d = dst_p.reshape(-1, CHUNK)
    rps = e_pad // 8
    rw = rps // (QS * CHUNK)
    jpc = _pick_jpc(rw)

    ea_t = edge_attr.astype(f32).T
    x_t = x.astype(f32).T
    w1_t = W1.astype(f32).T
    bf16 = jnp.bfloat16
    w2f = W2.astype(f32).reshape(d, d, d).transpose(2, 1, 0).reshape(
        d, d * d).astype(bf16)
    b2m = b2.astype(f32).reshape(d, d).T
    root_t = root.astype(f32).T
    bias128 = jnp.tile(bias.astype(f32), 8).reshape(1, 8 * d)

    mesh = plsc.VectorSubcoreMesh(core_axis_name="c", subcore_axis_name="s")
    gather = pl.kernel(
        functools.partial(_gather_body, rw, jpc, d),
        out_type=jax.ShapeDtypeStruct((rps, 8 * d), f32),
        mesh=mesh,
        scratch_types=[
            pltpu.VMEM((jpc, CHUNK), jnp.int32),
            pltpu.VMEM((jpc * CHUNK, d), f32),
            pltpu.SemaphoreType.DMA,
        ],
        compiler_params=pltpu.CompilerParams(use_tc_tiling_on_sc=False),
    )
    xjp = gather(x.astype(f32), src2d)

    nblk = e_pad // BT
    wb = BT // 8
    last_ea_blk = (e - 1) // wb
    def _ea_spec(kk):
        return pl.BlockSpec(
            (de, wb), lambda b: (0, jnp.minimum(nblk * kk + b, last_ea_blk)))
    msgp = pl.pallas_call(
        _msg_body_flat,
        grid=(nblk,),
        in_specs=[_ea_spec(kk) for kk in range(8)] + [
            pl.BlockSpec((wb, 8 * d), lambda b: (b, 0)),
            pl.BlockSpec((d, de), lambda b: (0, 0)),
            pl.BlockSpec((d, 1), lambda b: (0, 0)),
            pl.BlockSpec((d, d * d), lambda b: (0, 0)),
            pl.BlockSpec((d, d), lambda b: (0, 0)),
            pl.BlockSpec(memory_space=pltpu.MemorySpace.SMEM),
        ],
        out_specs=pl.BlockSpec((wb, 8 * d), lambda b: (b, 0)),
        out_shape=jax.ShapeDtypeStruct((rps, 8 * d), f32),
    )(*([ea_t] * 8), xjp, w1_t, b1.astype(f32).reshape(d, 1), w2f, b2m,
      jnp.asarray(a1, f32).reshape(1, 1))

    scatter = pl.kernel(
        functools.partial(_scatter_body, n_pad, rw, jpc, d),
        out_type=(jax.ShapeDtypeStruct((n_pad, d), f32),
                  jax.ShapeDtypeStruct((n_pad, d), f32)),
        mesh=mesh,
        scratch_types=[
            pltpu.VMEM((jpc, CHUNK), jnp.int32),
            pltpu.VMEM((jpc * CHUNK, d), f32),
            pltpu.VMEM_SHARED((n_pad, d), f32),
        ],
        compiler_params=pltpu.CompilerParams(use_tc_tiling_on_sc=False),
    )
    p0, p1 = scatter(msgp, dst2d, jnp.zeros((n_pad, d), f32))
    p0p = p0.reshape(n_pad // 8, 8 * d)
    p1p = p1.reshape(n_pad // 8, 8 * d)

    outp = pl.pallas_call(
        _final_body,
        grid=(n_pad // NB,),
        in_specs=[
            pl.BlockSpec((NB // 8, 8 * d), lambda b: (b, 0)),
            pl.BlockSpec((NB // 8, 8 * d), lambda b: (b, 0)),
            pl.BlockSpec((d, NB), lambda b: (0, b)),
            pl.BlockSpec((d, d), lambda b: (0, 0)),
            pl.BlockSpec((1, 8 * d), lambda b: (0, 0)),
            pl.BlockSpec(memory_space=pltpu.MemorySpace.SMEM),
        ],
        out_specs=pl.BlockSpec((NB // 8, 8 * d), lambda b: (b, 0)),
        out_shape=jax.ShapeDtypeStruct((n_pad // 8, 8 * d), f32),
    )(p0p, p1p, x_t, root_t, bias128, jnp.asarray(a2, f32).reshape(1, 1))

    out = outp.reshape(n_pad // NB, NB // 8, 8, d).swapaxes(1, 2).reshape(-1, d)
    return out[:n]


# speedup vs baseline: 16.9076x; 1.1746x over previous
import functools

import jax
import jax.numpy as jnp
from jax import lax
from jax.experimental import pallas as pl
from jax.experimental.pallas import tpu as pltpu
from jax.experimental.pallas import tpu_sc as plsc

NC = 2
NS = 16
NW = NC * NS
QS = 4
CHUNK = 128
BT = 8192
NB = 4096


def _pick_jpc(rw):
    for j in (14, 8, 7, 4, 2, 1):
        if rw % j == 0:
            return j
    return 1


def _gather_body(rw, jpc, d, half, x_hbm, idx_hbm, out_hbm, idx_v, rows_v, sem):
    c = lax.axis_index("c")
    s = lax.axis_index("s")
    wid = s * NC + c
    k = wid // QS
    q = wid % QS
    rps_rows = idx_hbm.shape[0] // 8
    row0 = rps_rows * k + half * (rps_rows // 2) + rw * q
    rout0 = rw * CHUNK * q

    def outer(i, carry):
        r = row0 + i * jpc
        pltpu.sync_copy(idx_hbm.at[pl.ds(r, jpc)], idx_v)
        copies = []
        for j in range(jpc):
            copies.append(
                pltpu.async_copy(
                    x_hbm.at[idx_v.at[j]],
                    rows_v.at[pl.ds(j * CHUNK, CHUNK)],
                    sem,
                )
            )
        for cp in copies:
            cp.wait()
        pltpu.sync_copy(
            rows_v,
            out_hbm.at[pl.ds(rout0 + i * jpc * CHUNK, jpc * CHUNK),
                       pl.ds(d * k, d)],
        )
        return carry

    lax.fori_loop(0, rw // jpc, outer, 0)


def _scatter_body(n_pad, rw, jpc, d, half, msg_hbm, dst_hbm, zeros_hbm,
                  out0_hbm, out1_hbm, idx_v, msg_v, acc):
    c = lax.axis_index("c")
    s = lax.axis_index("s")
    wid = s * NC + c
    k = wid // QS
    q = wid % QS
    rps_rows = dst_hbm.shape[0] // 8
    row0 = rps_rows * k + half * (rps_rows // 2) + rw * q
    rin0 = rw * CHUNK * q
    zr = n_pad // NS

    pltpu.sync_copy(zeros_hbm.at[pl.ds(s * zr, zr)], acc.at[pl.ds(s * zr, zr)])
    plsc.subcore_barrier()

    def outer(i, carry):
        pltpu.sync_copy(dst_hbm.at[pl.ds(row0 + i * jpc, jpc)], idx_v)
        pltpu.sync_copy(
            msg_hbm.at[pl.ds(rin0 + i * jpc * CHUNK, jpc * CHUNK),
                       pl.ds(d * k, d)],
            msg_v,
        )
        for j in range(jpc):
            pltpu.sync_copy(
                msg_v.at[pl.ds(j * CHUNK, CHUNK)],
                acc.at[idx_v.at[j]],
                add=True,
            )
        return carry

    lax.fori_loop(0, rw // jpc, outer, 0)
    plsc.subcore_barrier()

    @pl.when(c == 0)
    def _():
        pltpu.sync_copy(acc.at[pl.ds(s * zr, zr)], out0_hbm.at[pl.ds(s * zr, zr)])

    @pl.when(c == 1)
    def _():
        pltpu.sync_copy(acc.at[pl.ds(s * zr, zr)], out1_hbm.at[pl.ds(s * zr, zr)])


def _unpack_to_featmajor(packed):
    a = packed.T
    return jnp.concatenate([a[16 * k:16 * (k + 1), :] for k in range(8)], axis=1)


def _pack_from_featmajor(fm):
    w = fm.shape[1] // 8
    stacked = jnp.concatenate(
        [fm[:, w * k:w * (k + 1)] for k in range(8)], axis=0)
    return stacked.T


def _msg_body(ea_refs, xjp_ref, w1t_ref, b1_ref, w2f_ref, b2m_ref,
              a1_ref, out_ref):
    a1 = a1_ref[0, 0]
    f32 = jnp.float32
    bf16 = jnp.bfloat16
    ea_t = jnp.concatenate([r[...] for r in ea_refs], axis=1)
    d = ea_t.shape[0]
    ht = jnp.dot(w1t_ref[...], ea_t, preferred_element_type=f32)
    ht = ht + b1_ref[...]
    ht = jnp.where(ht >= 0, ht, a1 * ht)
    xj_t = _unpack_to_featmajor(xjp_ref[...])
    ht16 = ht.astype(bf16)
    x16 = xj_t.astype(bf16)
    zh = jnp.concatenate([ht16] * d, axis=0)
    zx = jnp.concatenate(
        [jnp.broadcast_to(x16[i:i + 1, :], ht16.shape) for i in range(d)],
        axis=0)
    msg_t = (jnp.dot(w2f_ref[...], zh * zx, preferred_element_type=f32)
             + jnp.dot(b2m_ref[...], xj_t, preferred_element_type=f32))
    out_ref[...] = _pack_from_featmajor(msg_t)


def _msg_body_flat(*refs):
    return _msg_body(refs[:8], *refs[8:])


def _final_body(p0_ref, p1_ref, p2_ref, p3_ref, xt_ref, roott_ref, bias_ref,
                a2_ref, out_ref):
    a2 = a2_ref[0, 0]
    xr_t = jnp.dot(roott_ref[...], xt_ref[...],
                   preferred_element_type=jnp.float32)
    v = (p0_ref[...] + p1_ref[...] + p2_ref[...] + p3_ref[...]
         + _pack_from_featmajor(xr_t) + bias_ref[...])
    out_ref[...] = jnp.where(v >= 0, v, a2 * v)


def kernel(x, edge_index, edge_attr, W1, b1, a1, W2, b2, root, bias, a2):
    n, d = x.shape
    e = edge_index.shape[1]
    de = edge_attr.shape[1]
    f32 = jnp.float32

    e_pad = ((e + BT - 1) // BT) * BT
    n_pad = ((n + NB - 1) // NB) * NB
    pad = e_pad - e
    src = edge_index[0].astype(jnp.int32)
    dst = edge_index[1].astype(jnp.int32)
    wn = NB // 8
    dblk, dloc = dst // NB, dst % NB
    dst_row = dblk * NB + (dloc % wn) * 8 + dloc // wn
    src_p = jnp.concatenate([src, jnp.zeros((pad,), jnp.int32)])
    mloc = (n_pad - 1) % NB
    dump_row = jnp.int32((n_pad - 1) // NB * NB + (mloc % wn) * 8 + mloc // wn)
    dst_p = jnp.concatenate([dst_row, jnp.full((pad,), dump_row, jnp.int32)])
    src2d = src_p.reshape(-1, CHUNK)
    dst2d = dst_p.reshape(-1, CHUNK)
    rps = e_pad // 8
    rw = rps // (QS * CHUNK)
    jpc = _pick_jpc(rw)

    ea_t = edge_attr.astype(f32).T
    x_t = x.astype(f32).T
    w1_t = W1.astype(f32).T
    bf16 = jnp.bfloat16
    w2f = W2.astype(f32).reshape(d, d, d).transpose(2, 1, 0).reshape(
        d, d * d).astype(bf16)
    b2m = b2.astype(f32).reshape(d, d).T
    root_t = root.astype(f32).T
    bias128 = jnp.tile(bias.astype(f32), 8).reshape(1, 8 * d)

    mesh = plsc.VectorSubcoreMesh(core_axis_name="c", subcore_axis_name="s")
    rwh = rw // 2
    jpch = _pick_jpc(rwh)
    rpsh = rps // 2
    nblk = e_pad // BT
    nblk_h = nblk // 2
    wb = BT // 8
    last_ea_blk = (e - 1) // wb
    x_lin = x.astype(f32)
    zeros = jnp.zeros((n_pad, d), f32)
    a1s = jnp.asarray(a1, f32).reshape(1, 1)
    b1c = b1.astype(f32).reshape(d, 1)

    def gather_half(h):
        g = pl.kernel(
            functools.partial(_gather_body, rwh, jpch, d, h),
            out_type=jax.ShapeDtypeStruct((rpsh, 8 * d), f32),
            mesh=mesh,
            scratch_types=[
                pltpu.VMEM((jpch, CHUNK), jnp.int32),
                pltpu.VMEM((jpch * CHUNK, d), f32),
                pltpu.SemaphoreType.DMA,
            ],
            compiler_params=pltpu.CompilerParams(use_tc_tiling_on_sc=False),
        )
        return g(x_lin, src2d)

    def msg_half(h, xjp_h):
        def _ea_spec(kk):
            return pl.BlockSpec(
                (de, wb),
                lambda b: (0, jnp.minimum(nblk * kk + nblk_h * h + b,
                                          last_ea_blk)))
        return pl.pallas_call(
            _msg_body_flat,
            grid=(nblk_h,),
            in_specs=[_ea_spec(kk) for kk in range(8)] + [
                pl.BlockSpec((wb, 8 * d), lambda b: (b, 0)),
                pl.BlockSpec((d, de), lambda b: (0, 0)),
                pl.BlockSpec((d, 1), lambda b: (0, 0)),
                pl.BlockSpec((d, d * d), lambda b: (0, 0)),
                pl.BlockSpec((d, d), lambda b: (0, 0)),
                pl.BlockSpec(memory_space=pltpu.MemorySpace.SMEM),
            ],
            out_specs=pl.BlockSpec((wb, 8 * d), lambda b: (b, 0)),
            out_shape=jax.ShapeDtypeStruct((rpsh, 8 * d), f32),
        )(*([ea_t] * 8), xjp_h, w1_t, b1c, w2f, b2m, a1s)

    def scatter_half(h, msgp_h):
        sc = pl.kernel(
            functools.partial(_scatter_body, n_pad, rwh, jpch, d, h),
            out_type=(jax.ShapeDtypeStruct((n_pad, d), f32),
                      jax.ShapeDtypeStruct((n_pad, d), f32)),
            mesh=mesh,
            scratch_types=[
                pltpu.VMEM((jpch, CHUNK), jnp.int32),
                pltpu.VMEM((jpch * CHUNK, d), f32),
                pltpu.VMEM_SHARED((n_pad, d), f32),
            ],
            compiler_params=pltpu.CompilerParams(use_tc_tiling_on_sc=False),
        )
        return sc(msgp_h, dst2d, zeros)

    xjp_a = gather_half(0)
    xjp_b = gather_half(1)
    msg_a = msg_half(0, xjp_a)
    msg_b = msg_half(1, xjp_b)
    p0a, p1a = scatter_half(0, msg_a)
    p0b, p1b = scatter_half(1, msg_b)
    parts = [p.reshape(n_pad // 8, 8 * d) for p in (p0a, p1a, p0b, p1b)]

    outp = pl.pallas_call(
        _final_body,
        grid=(n_pad // NB,),
        in_specs=[pl.BlockSpec((NB // 8, 8 * d), lambda b: (b, 0))
                  for _ in range(4)] + [
            pl.BlockSpec((d, NB), lambda b: (0, b)),
            pl.BlockSpec((d, d), lambda b: (0, 0)),
            pl.BlockSpec((1, 8 * d), lambda b: (0, 0)),
            pl.BlockSpec(memory_space=pltpu.MemorySpace.SMEM),
        ],
        out_specs=pl.BlockSpec((NB // 8, 8 * d), lambda b: (b, 0)),
        out_shape=jax.ShapeDtypeStruct((n_pad // 8, 8 * d), f32),
    )(*parts, x_t, root_t, bias128, jnp.asarray(a2, f32).reshape(1, 1))

    out = outp.reshape(n_pad // NB, NB // 8, 8, d).swapaxes(1, 2).reshape(-1, d)
    return out[:n]


# speedup vs baseline: 16.9967x; 1.0053x over previous
import functools

import jax
import jax.numpy as jnp
from jax import lax
from jax.experimental import pallas as pl
from jax.experimental.pallas import tpu as pltpu
from jax.experimental.pallas import tpu_sc as plsc

NC = 2
NS = 16
NW = NC * NS
QS = 4
CHUNK = 128
BT = 8192
NB = 4096


def _pick_jpc(rw):
    for j in (14, 8, 7, 4, 2, 1):
        if rw % j == 0:
            return j
    return 1


def _gather_body(rw, jpc, d, hoff, x_hbm, idx_hbm, out_hbm, idx_v, rows_v, sem):
    c = lax.axis_index("c")
    s = lax.axis_index("s")
    wid = s * NC + c
    k = wid // QS
    q = wid % QS
    rps_rows = idx_hbm.shape[0] // 8
    row0 = rps_rows * k + hoff + rw * q
    rout0 = rw * CHUNK * q

    def outer(i, carry):
        r = row0 + i * jpc
        pltpu.sync_copy(idx_hbm.at[pl.ds(r, jpc)], idx_v)
        copies = []
        for j in range(jpc):
            copies.append(
                pltpu.async_copy(
                    x_hbm.at[idx_v.at[j]],
                    rows_v.at[pl.ds(j * CHUNK, CHUNK)],
                    sem,
                )
            )
        for cp in copies:
            cp.wait()
        pltpu.sync_copy(
            rows_v,
            out_hbm.at[pl.ds(rout0 + i * jpc * CHUNK, jpc * CHUNK),
                       pl.ds(d * k, d)],
        )
        return carry

    lax.fori_loop(0, rw // jpc, outer, 0)


def _scatter_body(n_pad, rw, jpc, d, hoffs, *refs):
    nm = len(hoffs)
    msg_refs = refs[:nm]
    (dst_hbm, zeros_hbm, out0_hbm, out1_hbm, idx_v, msg_v, acc) = refs[nm:]
    c = lax.axis_index("c")
    s = lax.axis_index("s")
    wid = s * NC + c
    k = wid // QS
    q = wid % QS
    rps_rows = dst_hbm.shape[0] // 8
    rin0 = rw * CHUNK * q
    zr = n_pad // NS

    pltpu.sync_copy(zeros_hbm.at[pl.ds(s * zr, zr)], acc.at[pl.ds(s * zr, zr)])
    plsc.subcore_barrier()

    for msg_hbm, hoff in zip(msg_refs, hoffs):
        row0 = rps_rows * k + hoff + rw * q

        def outer(i, carry):
            pltpu.sync_copy(dst_hbm.at[pl.ds(row0 + i * jpc, jpc)], idx_v)
            pltpu.sync_copy(
                msg_hbm.at[pl.ds(rin0 + i * jpc * CHUNK, jpc * CHUNK),
                           pl.ds(d * k, d)],
                msg_v,
            )
            for j in range(jpc):
                pltpu.sync_copy(
                    msg_v.at[pl.ds(j * CHUNK, CHUNK)],
                    acc.at[idx_v.at[j]],
                    add=True,
                )
            return carry

        lax.fori_loop(0, rw // jpc, outer, 0)
    plsc.subcore_barrier()

    @pl.when(c == 0)
    def _():
        pltpu.sync_copy(acc.at[pl.ds(s * zr, zr)], out0_hbm.at[pl.ds(s * zr, zr)])

    @pl.when(c == 1)
    def _():
        pltpu.sync_copy(acc.at[pl.ds(s * zr, zr)], out1_hbm.at[pl.ds(s * zr, zr)])


def _unpack_to_featmajor(packed):
    a = packed.T
    return jnp.concatenate([a[16 * k:16 * (k + 1), :] for k in range(8)], axis=1)


def _pack_from_featmajor(fm):
    w = fm.shape[1] // 8
    stacked = jnp.concatenate(
        [fm[:, w * k:w * (k + 1)] for k in range(8)], axis=0)
    return stacked.T


def _msg_body(ea_refs, xjp_ref, w1t_ref, b1_ref, w2f_ref, b2m_ref,
              a1_ref, out_ref):
    a1 = a1_ref[0, 0]
    f32 = jnp.float32
    bf16 = jnp.bfloat16
    ea_t = jnp.concatenate([r[...] for r in ea_refs], axis=1)
    d = ea_t.shape[0]
    ht = jnp.dot(w1t_ref[...], ea_t, preferred_element_type=f32)
    ht = ht + b1_ref[...]
    ht = jnp.where(ht >= 0, ht, a1 * ht)
    xj_t = _unpack_to_featmajor(xjp_ref[...])
    ht16 = ht.astype(bf16)
    x16 = xj_t.astype(bf16)
    zh = jnp.concatenate([ht16] * d, axis=0)
    zx = jnp.concatenate(
        [jnp.broadcast_to(x16[i:i + 1, :], ht16.shape) for i in range(d)],
        axis=0)
    msg_t = (jnp.dot(w2f_ref[...], zh * zx, preferred_element_type=f32)
             + jnp.dot(b2m_ref[...], xj_t, preferred_element_type=f32))
    out_ref[...] = _pack_from_featmajor(msg_t)


def _msg_body_flat(*refs):
    return _msg_body(refs[:8], *refs[8:])


def _final_body(nparts, *refs):
    p_refs = refs[:nparts]
    xt_ref, roott_ref, bias_ref, a2_ref, out_ref = refs[nparts:]
    a2 = a2_ref[0, 0]
    xr_t = jnp.dot(roott_ref[...], xt_ref[...],
                   preferred_element_type=jnp.float32)
    v = _pack_from_featmajor(xr_t) + bias_ref[...]
    for p in p_refs:
        v = v + p[...]
    out_ref[...] = jnp.where(v >= 0, v, a2 * v)


def kernel(x, edge_index, edge_attr, W1, b1, a1, W2, b2, root, bias, a2):
    n, d = x.shape
    e = edge_index.shape[1]
    de = edge_attr.shape[1]
    f32 = jnp.float32

    e_pad = ((e + BT - 1) // BT) * BT
    n_pad = ((n + NB - 1) // NB) * NB
    pad = e_pad - e
    src = edge_index[0].astype(jnp.int32)
    dst = edge_index[1].astype(jnp.int32)
    wn = NB // 8
    dblk, dloc = dst // NB, dst % NB
    dst_row = dblk * NB + (dloc % wn) * 8 + dloc // wn
    src_p = jnp.concatenate([src, jnp.zeros((pad,), jnp.int32)])
    mloc = (n_pad - 1) % NB
    dump_row = jnp.int32((n_pad - 1) // NB * NB + (mloc % wn) * 8 + mloc // wn)
    dst_p = jnp.concatenate([dst_row, jnp.full((pad,), dump_row, jnp.int32)])
    src2d = src_p.reshape(-1, CHUNK)
    dst2d = dst_p.reshape(-1, CHUNK)
    rps = e_pad // 8
    rw = rps // (QS * CHUNK)
    jpc = _pick_jpc(rw)

    ea_t = edge_attr.astype(f32).T
    x_t = x.astype(f32).T
    w1_t = W1.astype(f32).T
    bf16 = jnp.bfloat16
    w2f = W2.astype(f32).reshape(d, d, d).transpose(2, 1, 0).reshape(
        d, d * d).astype(bf16)
    b2m = b2.astype(f32).reshape(d, d).T
    root_t = root.astype(f32).T
    bias128 = jnp.tile(bias.astype(f32), 8).reshape(1, 8 * d)

    nsplit = 7 if (rw % 7 == 0 and (e_pad // BT) % 7 == 0) else 2
    mesh = plsc.VectorSubcoreMesh(core_axis_name="c", subcore_axis_name="s")
    rwh = rw // nsplit
    jpch = _pick_jpc(rwh)
    rpsh = rps // nsplit
    nblk = e_pad // BT
    nblk_h = nblk // nsplit
    wb = BT // 8
    last_ea_blk = (e - 1) // wb
    x_lin = x.astype(f32)
    zeros = jnp.zeros((n_pad, d), f32)
    a1s = jnp.asarray(a1, f32).reshape(1, 1)
    b1c = b1.astype(f32).reshape(d, 1)

    def gather_half(h):
        g = pl.kernel(
            functools.partial(_gather_body, rwh, jpch, d, h * rwh * QS),
            out_type=jax.ShapeDtypeStruct((rpsh, 8 * d), f32),
            mesh=mesh,
            scratch_types=[
                pltpu.VMEM((jpch, CHUNK), jnp.int32),
                pltpu.VMEM((jpch * CHUNK, d), f32),
                pltpu.SemaphoreType.DMA,
            ],
            compiler_params=pltpu.CompilerParams(use_tc_tiling_on_sc=False),
        )
        return g(x_lin, src2d)

    def msg_half(h, xjp_h):
        def _ea_spec(kk):
            return pl.BlockSpec(
                (de, wb),
                lambda b: (0, jnp.minimum(nblk * kk + nblk_h * h + b,
                                          last_ea_blk)))
        return pl.pallas_call(
            _msg_body_flat,
            grid=(nblk_h,),
            in_specs=[_ea_spec(kk) for kk in range(8)] + [
                pl.BlockSpec((wb, 8 * d), lambda b: (b, 0)),
                pl.BlockSpec((d, de), lambda b: (0, 0)),
                pl.BlockSpec((d, 1), lambda b: (0, 0)),
                pl.BlockSpec((d, d * d), lambda b: (0, 0)),
                pl.BlockSpec((d, d), lambda b: (0, 0)),
                pl.BlockSpec(memory_space=pltpu.MemorySpace.SMEM),
            ],
            out_specs=pl.BlockSpec((wb, 8 * d), lambda b: (b, 0)),
            out_shape=jax.ShapeDtypeStruct((rpsh, 8 * d), f32),
        )(*([ea_t] * 8), xjp_h, w1_t, b1c, w2f, b2m, a1s)

    def scatter_group(hs, msg_slices):
        sc = pl.kernel(
            functools.partial(_scatter_body, n_pad, rwh, jpch, d,
                              tuple(h * rwh * QS for h in hs)),
            out_type=(jax.ShapeDtypeStruct((n_pad, d), f32),
                      jax.ShapeDtypeStruct((n_pad, d), f32)),
            mesh=mesh,
            scratch_types=[
                pltpu.VMEM((jpch, CHUNK), jnp.int32),
                pltpu.VMEM((jpch * CHUNK, d), f32),
                pltpu.VMEM_SHARED((n_pad, d), f32),
            ],
            compiler_params=pltpu.CompilerParams(use_tc_tiling_on_sc=False),
        )
        return sc(*msg_slices, dst2d, zeros)

    xjps = [gather_half(h) for h in range(nsplit)]
    msgs = [msg_half(h, xjps[h]) for h in range(nsplit)]
    cut = (nsplit + 1) // 2
    pairs = [scatter_group(range(0, cut), msgs[:cut]),
             scatter_group(range(cut, nsplit), msgs[cut:])]

    parts = [p.reshape(n_pad // 8, 8 * d) for pr in pairs for p in pr]

    outp = pl.pallas_call(
        functools.partial(_final_body, len(parts)),
        grid=(n_pad // NB,),
        in_specs=[pl.BlockSpec((NB // 8, 8 * d), lambda b: (b, 0))
                  for _ in parts] + [
            pl.BlockSpec((d, NB), lambda b: (0, b)),
            pl.BlockSpec((d, d), lambda b: (0, 0)),
            pl.BlockSpec((1, 8 * d), lambda b: (0, 0)),
            pl.BlockSpec(memory_space=pltpu.MemorySpace.SMEM),
        ],
        out_specs=pl.BlockSpec((NB // 8, 8 * d), lambda b: (b, 0)),
        out_shape=jax.ShapeDtypeStruct((n_pad // 8, 8 * d), f32),
    )(*parts, x_t, root_t, bias128, jnp.asarray(a2, f32).reshape(1, 1))

    out = outp.reshape(n_pad // NB, NB // 8, 8, d).swapaxes(1, 2).reshape(-1, d)
    return out[:n]


# speedup vs baseline: 17.5915x; 1.0350x over previous
import functools

import jax
import jax.numpy as jnp
from jax import lax
from jax.experimental import pallas as pl
from jax.experimental.pallas import tpu as pltpu
from jax.experimental.pallas import tpu_sc as plsc

NC = 2
NS = 16
NW = NC * NS
QS = 4
CHUNK = 128
BT = 8192
NB = 4096


def _pick_jpc(rw):
    for j in (14, 8, 7, 4, 2, 1):
        if rw % j == 0:
            return j
    return 1


def _gather_body(rw, jpc, d, hoff, x_hbm, idx_hbm, out_hbm, idx_v, rows_v, sem):
    c = lax.axis_index("c")
    s = lax.axis_index("s")
    wid = s * NC + c
    k = wid // QS
    q = wid % QS
    rps_rows = idx_hbm.shape[0] // 8
    row0 = rps_rows * k + hoff + rw * q
    rout0 = rw * CHUNK * q

    def outer(i, carry):
        r = row0 + i * jpc
        pltpu.sync_copy(idx_hbm.at[pl.ds(r, jpc)], idx_v)
        copies = []
        for j in range(jpc):
            copies.append(
                pltpu.async_copy(
                    x_hbm.at[idx_v.at[j]],
                    rows_v.at[pl.ds(j * CHUNK, CHUNK)],
                    sem,
                )
            )
        for cp in copies:
            cp.wait()
        pltpu.sync_copy(
            rows_v,
            out_hbm.at[pl.ds(rout0 + i * jpc * CHUNK, jpc * CHUNK),
                       pl.ds(d * k, d)],
        )
        return carry

    lax.fori_loop(0, rw // jpc, outer, 0)


def _scatter_body(n_pad, rw, jpc, d, hoffs, *refs):
    nm = len(hoffs)
    msg_refs = refs[:nm]
    (dst_hbm, zeros_hbm, out0_hbm, out1_hbm, idx_v, msg_v, acc) = refs[nm:]
    c = lax.axis_index("c")
    s = lax.axis_index("s")
    wid = s * NC + c
    k = wid // QS
    q = wid % QS
    rps_rows = dst_hbm.shape[0] // 8
    rin0 = rw * CHUNK * q
    zr = n_pad // NS

    pltpu.sync_copy(zeros_hbm.at[pl.ds(s * zr, zr)], acc.at[pl.ds(s * zr, zr)])
    plsc.subcore_barrier()

    for msg_hbm, hoff in zip(msg_refs, hoffs):
        row0 = rps_rows * k + hoff + rw * q

        def outer(i, carry):
            pltpu.sync_copy(dst_hbm.at[pl.ds(row0 + i * jpc, jpc)], idx_v)
            pltpu.sync_copy(
                msg_hbm.at[pl.ds(rin0 + i * jpc * CHUNK, jpc * CHUNK),
                           pl.ds(d * k, d)],
                msg_v,
            )
            for j in range(jpc):
                pltpu.sync_copy(
                    msg_v.at[pl.ds(j * CHUNK, CHUNK)],
                    acc.at[idx_v.at[j]],
                    add=True,
                )
            return carry

        lax.fori_loop(0, rw // jpc, outer, 0)
    plsc.subcore_barrier()

    @pl.when(c == 0)
    def _():
        pltpu.sync_copy(acc.at[pl.ds(s * zr, zr)], out0_hbm.at[pl.ds(s * zr, zr)])

    @pl.when(c == 1)
    def _():
        pltpu.sync_copy(acc.at[pl.ds(s * zr, zr)], out1_hbm.at[pl.ds(s * zr, zr)])


def _unpack_to_featmajor(packed):
    a = packed.T
    return jnp.concatenate([a[16 * k:16 * (k + 1), :] for k in range(8)], axis=1)


def _pack_from_featmajor(fm):
    w = fm.shape[1] // 8
    stacked = jnp.concatenate(
        [fm[:, w * k:w * (k + 1)] for k in range(8)], axis=0)
    return stacked.T


def _msg_body(ea_refs, xjp_ref, w1t_ref, b1_ref, w2f_ref, b2m_ref,
              a1_ref, out_ref):
    a1 = a1_ref[0, 0]
    f32 = jnp.float32
    bf16 = jnp.bfloat16
    ea_t = jnp.concatenate([r[...] for r in ea_refs], axis=1)
    d = ea_t.shape[0]
    ht = jnp.dot(w1t_ref[...], ea_t, preferred_element_type=f32)
    ht = ht + b1_ref[...]
    ht = jnp.where(ht >= 0, ht, a1 * ht)
    xj_t = _unpack_to_featmajor(xjp_ref[...])
    ht16 = ht.astype(bf16)
    x16 = xj_t.astype(bf16)
    zh = jnp.concatenate([ht16] * d, axis=0)
    zx = jnp.concatenate(
        [jnp.broadcast_to(x16[i:i + 1, :], ht16.shape) for i in range(d)],
        axis=0)
    msg_t = (jnp.dot(w2f_ref[...], zh * zx, preferred_element_type=f32)
             + jnp.dot(b2m_ref[...], xj_t, preferred_element_type=f32))
    out_ref[...] = _pack_from_featmajor(msg_t)


def _msg_body_flat(*refs):
    return _msg_body(refs[:8], *refs[8:])


def _xpack_body(xt_ref, out_ref):
    out_ref[...] = _pack_from_featmajor(xt_ref[...])


def _final_body(nparts, *refs):
    p_refs = refs[:nparts]
    xt_ref, roott_ref, bias_ref, a2_ref, out_ref = refs[nparts:]
    a2 = a2_ref[0, 0]
    xr_t = jnp.dot(roott_ref[...], xt_ref[...],
                   preferred_element_type=jnp.float32)
    v = _pack_from_featmajor(xr_t) + bias_ref[...]
    for p in p_refs:
        v = v + p[...]
    out_ref[...] = jnp.where(v >= 0, v, a2 * v)


def kernel(x, edge_index, edge_attr, W1, b1, a1, W2, b2, root, bias, a2):
    n, d = x.shape
    e = edge_index.shape[1]
    de = edge_attr.shape[1]
    f32 = jnp.float32

    e_pad = ((e + BT - 1) // BT) * BT
    n_pad = ((n + NB - 1) // NB) * NB
    pad = e_pad - e
    src = edge_index[0].astype(jnp.int32)
    dst = edge_index[1].astype(jnp.int32)
    wn = NB // 8
    dblk, dloc = dst // NB, dst % NB
    dst_row = dblk * NB + (dloc % wn) * 8 + dloc // wn
    sblk, sloc = src // NB, src % NB
    src_row = sblk * NB + (sloc % wn) * 8 + sloc // wn
    src_p = jnp.concatenate([src_row, jnp.zeros((pad,), jnp.int32)])
    mloc = (n_pad - 1) % NB
    dump_row = jnp.int32((n_pad - 1) // NB * NB + (mloc % wn) * 8 + mloc // wn)
    dst_p = jnp.concatenate([dst_row, jnp.full((pad,), dump_row, jnp.int32)])
    src2d = src_p.reshape(-1, CHUNK)
    dst2d = dst_p.reshape(-1, CHUNK)
    rps = e_pad // 8
    rw = rps // (QS * CHUNK)
    jpc = _pick_jpc(rw)

    ea_t = edge_attr.astype(f32).T
    x_t = x.astype(f32).T
    w1_t = W1.astype(f32).T
    bf16 = jnp.bfloat16
    w2f = W2.astype(f32).reshape(d, d, d).transpose(2, 1, 0).reshape(
        d, d * d).astype(bf16)
    b2m = b2.astype(f32).reshape(d, d).T
    root_t = root.astype(f32).T
    bias128 = jnp.tile(bias.astype(f32), 8).reshape(1, 8 * d)

    nsplit = 7 if (rw % 7 == 0 and (e_pad // BT) % 7 == 0) else 2
    mesh = plsc.VectorSubcoreMesh(core_axis_name="c", subcore_axis_name="s")
    rwh = rw // nsplit
    jpch = _pick_jpc(rwh)
    rpsh = rps // nsplit
    nblk = e_pad // BT
    nblk_h = nblk // nsplit
    wb = BT // 8
    last_ea_blk = (e - 1) // wb
    x_tab = pl.pallas_call(
        _xpack_body,
        grid=(n_pad // NB,),
        in_specs=[pl.BlockSpec((d, NB), lambda b: (0, b))],
        out_specs=pl.BlockSpec((NB // 8, 8 * d), lambda b: (b, 0)),
        out_shape=jax.ShapeDtypeStruct((n_pad // 8, 8 * d), f32),
    )(x_t).reshape(n_pad, d)
    zeros = jnp.zeros((n_pad, d), f32)
    a1s = jnp.asarray(a1, f32).reshape(1, 1)
    b1c = b1.astype(f32).reshape(d, 1)

    def gather_half(h):
        g = pl.kernel(
            functools.partial(_gather_body, rwh, jpch, d, h * rwh * QS),
            out_type=jax.ShapeDtypeStruct((rpsh, 8 * d), f32),
            mesh=mesh,
            scratch_types=[
                pltpu.VMEM((jpch, CHUNK), jnp.int32),
                pltpu.VMEM((jpch * CHUNK, d), f32),
                pltpu.SemaphoreType.DMA,
            ],
            compiler_params=pltpu.CompilerParams(use_tc_tiling_on_sc=False),
        )
        return g(x_tab, src2d)

    def msg_half(h, xjp_h):
        def _ea_spec(kk):
            return pl.BlockSpec(
                (de, wb),
                lambda b: (0, jnp.minimum(nblk * kk + nblk_h * h + b,
                                          last_ea_blk)))
        return pl.pallas_call(
            _msg_body_flat,
            grid=(nblk_h,),
            in_specs=[_ea_spec(kk) for kk in range(8)] + [
                pl.BlockSpec((wb, 8 * d), lambda b: (b, 0)),
                pl.BlockSpec((d, de), lambda b: (0, 0)),
                pl.BlockSpec((d, 1), lambda b: (0, 0)),
                pl.BlockSpec((d, d * d), lambda b: (0, 0)),
                pl.BlockSpec((d, d), lambda b: (0, 0)),
                pl.BlockSpec(memory_space=pltpu.MemorySpace.SMEM),
            ],
            out_specs=pl.BlockSpec((wb, 8 * d), lambda b: (b, 0)),
            out_shape=jax.ShapeDtypeStruct((rpsh, 8 * d), f32),
        )(*([ea_t] * 8), xjp_h, w1_t, b1c, w2f, b2m, a1s)

    def scatter_group(hs, msg_slices):
        sc = pl.kernel(
            functools.partial(_scatter_body, n_pad, rwh, jpch, d,
                              tuple(h * rwh * QS for h in hs)),
            out_type=(jax.ShapeDtypeStruct((n_pad, d), f32),
                      jax.ShapeDtypeStruct((n_pad, d), f32)),
            mesh=mesh,
            scratch_types=[
                pltpu.VMEM((jpch, CHUNK), jnp.int32),
                pltpu.VMEM((jpch * CHUNK, d), f32),
                pltpu.VMEM_SHARED((n_pad, d), f32),
            ],
            compiler_params=pltpu.CompilerParams(use_tc_tiling_on_sc=False),
        )
        return sc(*msg_slices, dst2d, zeros)

    xjps = [gather_half(h) for h in range(nsplit)]
    msgs = [msg_half(h, xjps[h]) for h in range(nsplit)]
    cut = (nsplit + 1) // 2
    pairs = [scatter_group(range(0, cut), msgs[:cut]),
             scatter_group(range(cut, nsplit), msgs[cut:])]

    parts = [p.reshape(n_pad // 8, 8 * d) for pr in pairs for p in pr]

    outp = pl.pallas_call(
        functools.partial(_final_body, len(parts)),
        grid=(n_pad // NB,),
        in_specs=[pl.BlockSpec((NB // 8, 8 * d), lambda b: (b, 0))
                  for _ in parts] + [
            pl.BlockSpec((d, NB), lambda b: (0, b)),
            pl.BlockSpec((d, d), lambda b: (0, 0)),
            pl.BlockSpec((1, 8 * d), lambda b: (0, 0)),
            pl.BlockSpec(memory_space=pltpu.MemorySpace.SMEM),
        ],
        out_specs=pl.BlockSpec((NB // 8, 8 * d), lambda b: (b, 0)),
        out_shape=jax.ShapeDtypeStruct((n_pad // 8, 8 * d), f32),
    )(*parts, x_t, root_t, bias128, jnp.asarray(a2, f32).reshape(1, 1))

    out = outp.reshape(n_pad // NB, NB // 8, 8, d).swapaxes(1, 2).reshape(-1, d)
    return out[:n]
